# Initial kernel scaffold; baseline (speedup 1.0000x reference)
import jax
import jax.numpy as jnp
from jax.experimental import pallas as pl


def kernel(dense_xyz, dense_feats, nei_inds, wn_w1, wn_b1, wn_g1, wn_be1, wn_w2, wn_b2, wn_g2, wn_be2, wn_w3, wn_b3, wn_g3, wn_be3, lin_w, lin_b, u2_w, u2_b, u2_g, u2_be, sc_w, sc_b, sc_g, sc_be):
    raise NotImplementedError("write your pallas kernel here")


# speedup vs baseline: 3.6125x; 3.6125x over previous
import functools

import jax
import jax.numpy as jnp
from jax.experimental import pallas as pl
from jax.experimental.pallas import tpu as pltpu
from jax.experimental.pallas import tpu_sc as plsc

_EPS = 1e-5
_GATHER_WIN = 128


def _sc_gather(feats, xyzp, idx2d, e_pad):
    n_feat = feats.shape[1]
    n_xyz = xyzp.shape[1]
    mesh = plsc.VectorSubcoreMesh(core_axis_name="core",
                                  subcore_axis_name="subcore")

    @functools.partial(
        pl.kernel,
        out_type=(
            jax.ShapeDtypeStruct((e_pad, n_feat), jnp.float32),
            jax.ShapeDtypeStruct((e_pad, n_xyz), jnp.float32),
        ),
        mesh=mesh,
        compiler_params=pltpu.CompilerParams(use_tc_tiling_on_sc=False),
    )
    def sc_kernel(feats_hbm, xyzp_hbm, idx_hbm, gf_hbm, gx_hbm):
        def body(idx_vmem, gf_vmem, gx_vmem):
            pltpu.sync_copy(feats_hbm.at[idx_vmem.at[0]], gf_vmem)
            pltpu.sync_copy(xyzp_hbm.at[idx_vmem.at[0]], gx_vmem)

        pltpu.emit_pipeline(
            body,
            grid=(e_pad // _GATHER_WIN,),
            in_specs=[
                pl.BlockSpec((1, _GATHER_WIN), lambda i: (0, i)),
            ],
            out_specs=[
                pl.BlockSpec((_GATHER_WIN, n_feat), lambda i: (i, 0)),
                pl.BlockSpec((_GATHER_WIN, n_xyz), lambda i: (i, 0)),
            ],
            core_axis_name=("core", "subcore"),
            dimension_semantics=(pltpu.PARALLEL,),
        )(idx_hbm, gf_hbm, gx_hbm)

    return sc_kernel(feats, xyzp, idx2d)


def _tc_body(n_blocks, n_k,
             gx_ref, gf_ref, xyz_ref, ft_ref,
             w1_ref, b1_ref, w2_ref, b2_ref, w3_ref, b3_ref,
             lw_ref, lb_ref, u2w_ref, u2b_ref, scw_ref, scb_ref,
             out_ref, loc_ref, m_ref):
    k = pl.program_id(1)
    loc = gx_ref[:, 0:3] - xyz_ref[...]
    loc_ref[0] = loc

    loc_t = loc.T
    h = b1_ref[...]
    for d in range(3):
        h = h + w1_ref[:, d:d + 1] * loc_t[d:d + 1, :]
    h = jnp.maximum(h, 0.0)
    h2 = b2_ref[...]
    for d in range(8):
        h2 = h2 + w2_ref[:, d:d + 1] * h[d:d + 1, :]
    h2 = jnp.maximum(h2, 0.0)
    wv = b3_ref[...]
    for d in range(8):
        wv = wv + w3_ref[:, d:d + 1] * h2[d:d + 1, :]
    wv = jnp.maximum(wv, 0.0)

    w_pk = wv.T
    gf = gf_ref[...]
    m_part = jnp.concatenate(
        [gf * w_pk[:, j:j + 1] for j in range(16)], axis=1)

    @pl.when(k == 0)
    def _():
        m_ref[...] = m_part

    @pl.when(k > 0)
    def _():
        m_ref[...] = m_ref[...] + m_part

    @pl.when(k == n_k - 1)
    def _():
        m = m_ref[...]
        o = jnp.dot(m, lw_ref[...], preferred_element_type=jnp.float32)
        o = jnp.maximum(o + lb_ref[...], 0.0)
        o = jnp.dot(o, u2w_ref[...], preferred_element_type=jnp.float32)
        o = o + u2b_ref[...]
        s = jnp.dot(ft_ref[...], scw_ref[...], preferred_element_type=jnp.float32)
        s = s + scb_ref[...]
        t = o + s
        out_ref[...] = jnp.where(t >= 0.0, t, 0.1 * t)


def kernel(dense_xyz, dense_feats, nei_inds,
           wn_w1, wn_b1, wn_g1, wn_be1,
           wn_w2, wn_b2, wn_g2, wn_be2,
           wn_w3, wn_b3, wn_g3, wn_be3,
           lin_w, lin_b,
           u2_w, u2_b, u2_g, u2_be,
           sc_w, sc_b, sc_g, sc_be):
    B, N, _ = dense_xyz.shape
    K = nei_inds.shape[2]
    Cin = dense_feats.shape[2]
    Cout = u2_w.shape[1]

    xyz = dense_xyz[0]
    feats = dense_feats[0]
    xyzp = jnp.pad(xyz, ((0, 0), (0, 13)))

    E = N * K
    chunk = 32 * _GATHER_WIN
    e_pad = ((E + chunk - 1) // chunk) * chunk
    idx_km = jnp.transpose(nei_inds[0], (1, 0)).reshape(-1).astype(jnp.int32)
    idx2d = jnp.pad(idx_km, (0, e_pad - E)).reshape(1, e_pad)

    gf_all, gx_all = _sc_gather(feats, xyzp, idx2d, e_pad)

    inv = 1.0 / jnp.sqrt(1.0 + _EPS)
    s1 = wn_g1 * inv
    w1t = (wn_w1 * s1[None, :]).T
    b1c = (wn_b1 * s1 + wn_be1)[:, None]
    s2 = wn_g2 * inv
    w2t = (wn_w2 * s2[None, :]).T
    b2c = (wn_b2 * s2 + wn_be2)[:, None]
    s3 = wn_g3 * inv
    w3t = (wn_w3 * s3[None, :]).T
    b3c = (wn_b3 * s3 + wn_be3)[:, None]
    wn_out = w3t.shape[0]
    linp = lin_w.reshape(Cin, wn_out, -1).transpose(1, 0, 2).reshape(
        Cin * wn_out, -1)
    lb = lin_b[None, :]
    su2 = u2_g * inv
    u2wf = u2_w * su2[None, :]
    u2bf = (u2_b * su2 + u2_be)[None, :]
    ssc = sc_g * inv
    scwf = sc_w * ssc[None, :]
    scbf = (sc_b * ssc + sc_be)[None, :]

    P = 1000
    n_blocks = N // P
    km = lambda pi, k: (k * n_blocks + pi, 0)
    fixed = lambda pi, k: (0, 0)

    out, loc_knd = pl.pallas_call(
        functools.partial(_tc_body, n_blocks, K),
        grid=(n_blocks, K),
        in_specs=[
            pl.BlockSpec((P, 16), km),
            pl.BlockSpec((P, Cin), km),
            pl.BlockSpec((P, 3), lambda pi, k: (pi, 0)),
            pl.BlockSpec((P, Cin), lambda pi, k: (pi, 0)),
            pl.BlockSpec(w1t.shape, fixed),
            pl.BlockSpec(b1c.shape, fixed),
            pl.BlockSpec(w2t.shape, fixed),
            pl.BlockSpec(b2c.shape, fixed),
            pl.BlockSpec(w3t.shape, fixed),
            pl.BlockSpec(b3c.shape, fixed),
            pl.BlockSpec(linp.shape, fixed),
            pl.BlockSpec(lb.shape, fixed),
            pl.BlockSpec(u2wf.shape, fixed),
            pl.BlockSpec(u2bf.shape, fixed),
            pl.BlockSpec(scwf.shape, fixed),
            pl.BlockSpec(scbf.shape, fixed),
        ],
        out_specs=[
            pl.BlockSpec((P, Cout), lambda pi, k: (pi, 0)),
            pl.BlockSpec((1, P, 3), lambda pi, k: (k, pi, 0)),
        ],
        out_shape=[
            jax.ShapeDtypeStruct((N, Cout), jnp.float32),
            jax.ShapeDtypeStruct((K, N, 3), jnp.float32),
        ],
        scratch_shapes=[pltpu.VMEM((P, Cin * wn_out), jnp.float32)],
        compiler_params=pltpu.CompilerParams(
            dimension_semantics=("parallel", "arbitrary")),
    )(gx_all, gf_all, xyz, feats,
      w1t, b1c, w2t, b2c, w3t, b3c,
      linp, lb, u2wf, u2bf, scwf, scbf)

    new_feat = out[None]
    weight_net_input = jnp.transpose(loc_knd, (1, 0, 2))[None]
    return new_feat, weight_net_input


# speedup vs baseline: 7.9098x; 2.1896x over previous
import functools

import jax
import jax.numpy as jnp
from jax.experimental import pallas as pl
from jax.experimental.pallas import tpu as pltpu
from jax.experimental.pallas import tpu_sc as plsc

_EPS = 1e-5
_GATHER_WIN = 128


def _sc_gather(feats, xyzp, idx2d, e_pad):
    n_feat = feats.shape[1]
    n_xyz = xyzp.shape[1]
    mesh = plsc.VectorSubcoreMesh(core_axis_name="core",
                                  subcore_axis_name="subcore")

    @functools.partial(
        pl.kernel,
        out_type=(
            jax.ShapeDtypeStruct((e_pad, n_feat), jnp.float32),
            jax.ShapeDtypeStruct((e_pad, n_xyz), jnp.float32),
        ),
        mesh=mesh,
        compiler_params=pltpu.CompilerParams(use_tc_tiling_on_sc=False),
    )
    def sc_kernel(feats_hbm, xyzp_hbm, idx_hbm, gf_hbm, gx_hbm):
        def body(idx_vmem, gf_vmem, gx_vmem):
            pltpu.sync_copy(feats_hbm.at[idx_vmem.at[0]], gf_vmem)
            pltpu.sync_copy(xyzp_hbm.at[idx_vmem.at[0]], gx_vmem)

        pltpu.emit_pipeline(
            body,
            grid=(e_pad // _GATHER_WIN,),
            in_specs=[
                pl.BlockSpec((1, _GATHER_WIN), lambda i: (0, i)),
            ],
            out_specs=[
                pl.BlockSpec((_GATHER_WIN, n_feat), lambda i: (i, 0)),
                pl.BlockSpec((_GATHER_WIN, n_xyz), lambda i: (i, 0)),
            ],
            core_axis_name=("core", "subcore"),
            dimension_semantics=(pltpu.PARALLEL,),
        )(idx_hbm, gf_hbm, gx_hbm)

    return sc_kernel(feats, xyzp, idx2d)


def _tc_body(n_blocks, n_k,
             gx_ref, gf_ref, xyz_ref, ft_ref,
             w1_ref, b1_ref, w2_ref, b2_ref, w3_ref, b3_ref,
             lw_ref, lb_ref, u2w_ref, u2b_ref, scw_ref, scb_ref,
             out_ref, loc_ref, m_ref):
    k = pl.program_id(1)
    loc = gx_ref[:, 0:3] - xyz_ref[...]
    loc_ref[0] = loc

    loc_t = loc.T
    h = b1_ref[...]
    for d in range(3):
        h = h + w1_ref[:, d:d + 1] * loc_t[d:d + 1, :]
    h = jnp.maximum(h, 0.0)
    h2 = b2_ref[...]
    for d in range(8):
        h2 = h2 + w2_ref[:, d:d + 1] * h[d:d + 1, :]
    h2 = jnp.maximum(h2, 0.0)
    wv = b3_ref[...]
    for d in range(8):
        wv = wv + w3_ref[:, d:d + 1] * h2[d:d + 1, :]
    wv = jnp.maximum(wv, 0.0)

    gf_t = gf_ref[...].T
    n_w = wv.shape[0]
    pieces = [jnp.broadcast_to(wv[j:j + 1, :], gf_t.shape) * gf_t
              for j in range(n_w)]

    @pl.when(k == 0)
    def _():
        for j in range(n_w):
            m_ref[j * 64:(j + 1) * 64, :] = pieces[j]

    @pl.when(k > 0)
    def _():
        for j in range(n_w):
            sl = slice(j * 64, (j + 1) * 64)
            m_ref[sl, :] = m_ref[sl, :] + pieces[j]

    @pl.when(k == n_k - 1)
    def _():
        m = m_ref[...]
        o = jnp.dot(lw_ref[...], m, preferred_element_type=jnp.float32)
        o = jnp.maximum(o + lb_ref[...], 0.0)
        o = jnp.dot(u2w_ref[...], o, preferred_element_type=jnp.float32)
        o = o + u2b_ref[...]
        s = jnp.dot(scw_ref[...], ft_ref[...].T,
                    preferred_element_type=jnp.float32)
        s = s + scb_ref[...]
        t = o + s
        t = jnp.where(t >= 0.0, t, 0.1 * t)
        out_ref[...] = t.T


def kernel(dense_xyz, dense_feats, nei_inds,
           wn_w1, wn_b1, wn_g1, wn_be1,
           wn_w2, wn_b2, wn_g2, wn_be2,
           wn_w3, wn_b3, wn_g3, wn_be3,
           lin_w, lin_b,
           u2_w, u2_b, u2_g, u2_be,
           sc_w, sc_b, sc_g, sc_be):
    B, N, _ = dense_xyz.shape
    K = nei_inds.shape[2]
    Cin = dense_feats.shape[2]
    Cout = u2_w.shape[1]

    xyz = dense_xyz[0]
    feats = dense_feats[0]
    xyzp = jnp.pad(xyz, ((0, 0), (0, 13)))

    E = N * K
    chunk = 32 * _GATHER_WIN
    e_pad = ((E + chunk - 1) // chunk) * chunk
    idx_km = jnp.transpose(nei_inds[0], (1, 0)).reshape(-1).astype(jnp.int32)
    idx2d = jnp.pad(idx_km, (0, e_pad - E)).reshape(1, e_pad)

    gf_all, gx_all = _sc_gather(feats, xyzp, idx2d, e_pad)

    inv = 1.0 / jnp.sqrt(1.0 + _EPS)
    s1 = wn_g1 * inv
    w1t = (wn_w1 * s1[None, :]).T
    b1c = (wn_b1 * s1 + wn_be1)[:, None]
    s2 = wn_g2 * inv
    w2t = (wn_w2 * s2[None, :]).T
    b2c = (wn_b2 * s2 + wn_be2)[:, None]
    s3 = wn_g3 * inv
    w3t = (wn_w3 * s3[None, :]).T
    b3c = (wn_b3 * s3 + wn_be3)[:, None]
    wn_out = w3t.shape[0]
    linp = lin_w.reshape(Cin, wn_out, -1).transpose(2, 1, 0).reshape(
        -1, Cin * wn_out)
    lb = lin_b[:, None]
    su2 = u2_g * inv
    u2wf = (u2_w * su2[None, :]).T
    u2bf = (u2_b * su2 + u2_be)[:, None]
    ssc = sc_g * inv
    scwf = (sc_w * ssc[None, :]).T
    scbf = (sc_b * ssc + sc_be)[:, None]

    P = 2000
    n_blocks = N // P
    km = lambda pi, k: (k * n_blocks + pi, 0)
    fixed = lambda pi, k: (0, 0)

    out, loc_knd = pl.pallas_call(
        functools.partial(_tc_body, n_blocks, K),
        grid=(n_blocks, K),
        in_specs=[
            pl.BlockSpec((P, 16), km),
            pl.BlockSpec((P, Cin), km),
            pl.BlockSpec((P, 3), lambda pi, k: (pi, 0)),
            pl.BlockSpec((P, Cin), lambda pi, k: (pi, 0)),
            pl.BlockSpec(w1t.shape, fixed),
            pl.BlockSpec(b1c.shape, fixed),
            pl.BlockSpec(w2t.shape, fixed),
            pl.BlockSpec(b2c.shape, fixed),
            pl.BlockSpec(w3t.shape, fixed),
            pl.BlockSpec(b3c.shape, fixed),
            pl.BlockSpec(linp.shape, fixed),
            pl.BlockSpec(lb.shape, fixed),
            pl.BlockSpec(u2wf.shape, fixed),
            pl.BlockSpec(u2bf.shape, fixed),
            pl.BlockSpec(scwf.shape, fixed),
            pl.BlockSpec(scbf.shape, fixed),
        ],
        out_specs=[
            pl.BlockSpec((P, Cout), lambda pi, k: (pi, 0)),
            pl.BlockSpec((1, P, 3), lambda pi, k: (k, pi, 0)),
        ],
        out_shape=[
            jax.ShapeDtypeStruct((N, Cout), jnp.float32),
            jax.ShapeDtypeStruct((K, N, 3), jnp.float32),
        ],
        scratch_shapes=[pltpu.VMEM((Cin * wn_out, P), jnp.float32)],
        compiler_params=pltpu.CompilerParams(
            dimension_semantics=("parallel", "arbitrary")),
    )(gx_all, gf_all, xyz, feats,
      w1t, b1c, w2t, b2c, w3t, b3c,
      linp, lb, u2wf, u2bf, scwf, scbf)

    new_feat = out[None]
    weight_net_input = jnp.transpose(loc_knd, (1, 0, 2))[None]
    return new_feat, weight_net_input
